# baseline (speedup 1.0000x reference)
import functools

import jax
import jax.numpy as jnp
from jax import lax
from jax.experimental import pallas as pl
from jax.experimental.pallas import tpu as pltpu
from jax.experimental.pallas import tpu_sc as plsc

_A = 1.5769434603113077
_B = 0.8950608779109733
_GAMMA = 1.0
_EPS = 1e-4
_LANES = 16


@functools.lru_cache(maxsize=None)
def _sc_dist2(B, K, D, N):
    info = plsc.get_sparse_core_info()
    NC, NS = info.num_cores, info.num_subcores
    NW = NC * NS
    BPW = B // NW
    PW = BPW * K
    BCH = 4
    CH = BCH * K
    NCH = BPW // BCH
    DC = D // _LANES
    D2 = 2 * D
    assert B % NW == 0 and BPW % BCH == 0 and D % _LANES == 0
    assert CH % 8 == 0 and CH <= 128 and N % 2 == 0
    assert 4 * BCH == _LANES and CH % _LANES == 0

    mesh = plsc.VectorSubcoreMesh(core_axis_name="c", subcore_axis_name="s")

    @functools.partial(
        pl.kernel,
        mesh=mesh,
        compiler_params=pltpu.CompilerParams(
            needs_layout_passes=False, use_tc_tiling_on_sc=True),
        out_type=jax.ShapeDtypeStruct((B * K,), jnp.float32),
        scratch_types=[
            pltpu.VMEM((BPW,), jnp.int32),
            pltpu.VMEM((BPW,), jnp.int32),
            pltpu.VMEM((BPW, D2), jnp.float32),
            pltpu.VMEM((NCH, CH), jnp.int32),
            pltpu.VMEM((NCH, CH), jnp.int32),
            pltpu.VMEM((4, CH, D2), jnp.float32),
            pltpu.VMEM((_LANES, _LANES), jnp.float32),
            pltpu.VMEM((PW,), jnp.float32),
            pltpu.SemaphoreType.DMA,
            pltpu.SemaphoreType.DMA,
            pltpu.SemaphoreType.DMA,
            pltpu.SemaphoreType.DMA,
            pltpu.SemaphoreType.DMA,
        ],
    )
    def dist2(emb_hbm, jh_hbm, jp_hbm, fh_hbm, fp_hbm, out_hbm,
              fh_v, fp_v, from_v, jh_v, jp_v, to_v, col_v, d2_v,
              sem_f, sem_a, sem_b, sem_c, sem_d):
        wid = lax.axis_index("s") * NC + lax.axis_index("c")
        base = pl.multiple_of(wid * BPW, 8)
        pltpu.sync_copy(fh_hbm.at[pl.ds(base, BPW)], fh_v)
        gf = pltpu.async_copy(emb_hbm.at[fh_v], from_v, sem_f)
        pltpu.sync_copy(fp_hbm.at[pl.ds(base, BPW)], fp_v)
        pltpu.sync_copy(jh_hbm.at[pl.ds(wid * NCH, NCH), :], jh_v)
        pltpu.sync_copy(jp_hbm.at[pl.ds(wid * NCH, NCH), :], jp_v)

        sems = (sem_a, sem_b, sem_c, sem_d)
        NBUF = 4

        def start_gather(c, i):
            return pltpu.async_copy(emb_hbm.at[jh_v.at[c]], to_v.at[i],
                                    sems[i])

        for i in range(NBUF):
            start_gather(i, i)
        gf.wait()

        rowid = lax.iota(jnp.int32, _LANES)
        NPG = CH // _LANES

        def load_to(i, r, off):
            return [to_v[i, r, pl.ds(off + cc * _LANES, _LANES)]
                    for cc in range(DC)]

        def load_from(b, off):
            return [from_v[b, pl.ds(off + cc * _LANES, _LANES)]
                    for cc in range(DC)]

        def outer(t, carry):
            c0 = t * NBUF
            fpv = fp_v[pl.ds(t * (NBUF * BCH), _LANES)]
            for i in range(NBUF):
                c = c0 + i
                pltpu.make_async_copy(
                    emb_hbm.at[jh_v.at[0]], to_v.at[i], sems[i]).wait()
                jpv = [jp_v[c, pl.ds(g * _LANES, _LANES)] for g in range(NPG)]
                f_cur = load_from(c * BCH, fpv[i * BCH])
                t_cur = load_to(i, 0, jpv[0][0])
                for r in range(CH):
                    if r + 1 < CH:
                        rn = r + 1
                        t_nxt = load_to(i, rn, jpv[rn // _LANES][rn % _LANES])
                        if rn % K == 0:
                            f_nxt = load_from(c * BCH + rn // K,
                                              fpv[i * BCH + rn // K])
                        else:
                            f_nxt = f_cur
                    sq = []
                    for cc in range(DC):
                        dlt = f_cur[cc] - t_cur[cc]
                        sq.append(dlt * dlt)
                    while len(sq) > 1:
                        sq = [a + b for a, b in zip(sq[0::2], sq[1::2])] + (
                            [sq[-1]] if len(sq) % 2 else [])
                    acc = sq[0]
                    plsc.store_scatter(
                        col_v, [rowid, jnp.full((_LANES,), r % _LANES,
                                                jnp.int32)], acc)
                    if r % _LANES == _LANES - 1:
                        g = r // _LANES
                        rowsum = [col_v[l, :] for l in range(_LANES)]
                        while len(rowsum) > 1:
                            rowsum = [a + b for a, b in
                                      zip(rowsum[0::2], rowsum[1::2])]
                        d2_v[pl.ds(c * CH + g * _LANES, _LANES)] = rowsum[0]
                    if r + 1 < CH:
                        f_cur, t_cur = f_nxt, t_nxt

                @pl.when(c + NBUF < NCH)
                def _():
                    start_gather(c + NBUF, i)
            return carry

        lax.fori_loop(0, NCH // NBUF, outer, 0)
        pltpu.sync_copy(d2_v, out_hbm.at[pl.ds(pl.multiple_of(base * K, 8), PW)])

    return dist2, CH


def _tc_loss_body(d2_ref, v_ref, o_ref):
    x = d2_ref[...]
    v = v_ref[...]
    u = jnp.exp(_B * jnp.log(x))
    m = 1.0 / (1.0 + _A * u)
    att = jnp.log(jnp.clip(m, _EPS, 1.0))
    rep = jnp.log(jnp.clip(1.0 - m, _EPS, 1.0))
    loss = -v * att - _GAMMA * (1.0 - v) * rep
    o_ref[...] = jnp.mean(loss, axis=-1)


def kernel(embedding, V, s, e, J):
    B, K = J.shape
    N, D = embedding.shape
    dist2, CH = _sc_dist2(B, K, D, N)
    start = jnp.minimum(jnp.asarray(s, jnp.int32), jnp.asarray(e, jnp.int32) - B)
    fidx = start + jnp.arange(B, dtype=jnp.int32)
    emb2 = embedding.reshape(N // 2, 2 * D)
    jh = (J >> 1).reshape(B * K // CH, CH)
    jp = ((J & 1) * D).reshape(B * K // CH, CH)
    fh = fidx >> 1
    fp = (fidx & 1) * D
    d2 = dist2(emb2, jh, jp, fh, fp).reshape(B, K)
    return pl.pallas_call(
        _tc_loss_body,
        out_shape=jax.ShapeDtypeStruct((B,), jnp.float32),
    )(d2, V)


# speedup vs baseline: 1.3293x; 1.3293x over previous
import functools

import jax
import jax.numpy as jnp
from jax import lax
from jax.experimental import pallas as pl
from jax.experimental.pallas import tpu as pltpu
from jax.experimental.pallas import tpu_sc as plsc

_A = 1.5769434603113077
_B = 0.8950608779109733
_GAMMA = 1.0
_EPS = 1e-4
_LANES = 16


@functools.lru_cache(maxsize=None)
def _sc_dist2(B, K, D, N):
    info = plsc.get_sparse_core_info()
    NC, NS = info.num_cores, info.num_subcores
    NW = NC * NS
    BPW = B // NW
    PW = BPW * K
    BCH = 4
    CH = BCH * K
    NCH = BPW // BCH
    DC = D // _LANES
    assert B % NW == 0 and BPW % BCH == 0 and D % _LANES == 0
    assert CH % 8 == 0 and CH <= 128

    mesh = plsc.VectorSubcoreMesh(core_axis_name="c", subcore_axis_name="s")

    @functools.partial(
        pl.kernel,
        mesh=mesh,
        compiler_params=pltpu.CompilerParams(
            needs_layout_passes=False, use_tc_tiling_on_sc=False),
        out_type=jax.ShapeDtypeStruct((B * K,), jnp.float32),
        scratch_types=[
            pltpu.VMEM((BPW,), jnp.int32),
            pltpu.VMEM((BPW, D), jnp.float32),
            pltpu.VMEM((NCH, CH), jnp.int32),
            pltpu.VMEM((4, CH, D), jnp.float32),
            pltpu.VMEM((_LANES, _LANES), jnp.float32),
            pltpu.VMEM((PW,), jnp.float32),
            pltpu.SemaphoreType.DMA,
            pltpu.SemaphoreType.DMA,
            pltpu.SemaphoreType.DMA,
            pltpu.SemaphoreType.DMA,
            pltpu.SemaphoreType.DMA,
        ],
    )
    def dist2(emb_hbm, j2_hbm, fidx_hbm, out_hbm,
              fidx_v, from_v, j_v, to_v, col_v, d2_v,
              sem_f, sem_a, sem_b, sem_c, sem_d):
        wid = lax.axis_index("s") * NC + lax.axis_index("c")
        base = pl.multiple_of(wid * BPW, 8)
        pltpu.sync_copy(fidx_hbm.at[pl.ds(base, BPW)], fidx_v)
        gf = pltpu.async_copy(emb_hbm.at[fidx_v], from_v, sem_f)
        pltpu.sync_copy(j2_hbm.at[pl.ds(wid * NCH, NCH), :], j_v)

        sems = (sem_a, sem_b, sem_c, sem_d)
        NBUF = 4

        def start_gather(c, i):
            return pltpu.async_copy(emb_hbm.at[j_v.at[c]], to_v.at[i], sems[i])

        for i in range(NBUF):
            start_gather(i, i)
        gf.wait()

        rowid = lax.iota(jnp.int32, _LANES)

        def load_to(i, r):
            return [to_v[i, r, pl.ds(cc * _LANES, _LANES)] for cc in range(DC)]

        def load_from(b):
            return [from_v[b, pl.ds(cc * _LANES, _LANES)] for cc in range(DC)]

        def outer(t, carry):
            c0 = t * NBUF
            for i in range(NBUF):
                c = c0 + i
                pltpu.make_async_copy(
                    emb_hbm.at[j_v.at[0]], to_v.at[i], sems[i]).wait()
                f_cur = load_from(c * BCH)
                t_cur = load_to(i, 0)
                for r in range(CH):
                    if r + 1 < CH:
                        t_nxt = load_to(i, r + 1)
                        if (r + 1) % K == 0:
                            f_nxt = load_from(c * BCH + (r + 1) // K)
                        else:
                            f_nxt = f_cur
                    sq = []
                    for cc in range(DC):
                        dlt = f_cur[cc] - t_cur[cc]
                        sq.append(dlt * dlt)
                    while len(sq) > 1:
                        sq = [a + b for a, b in zip(sq[0::2], sq[1::2])] + (
                            [sq[-1]] if len(sq) % 2 else [])
                    acc = sq[0]
                    plsc.store_scatter(
                        col_v, [rowid, jnp.full((_LANES,), r % _LANES,
                                                jnp.int32)], acc)
                    if r % _LANES == _LANES - 1:
                        g = r // _LANES
                        rowsum = [col_v[l, :] for l in range(_LANES)]
                        while len(rowsum) > 1:
                            rowsum = [a + b for a, b in
                                      zip(rowsum[0::2], rowsum[1::2])]
                        d2_v[pl.ds(c * CH + g * _LANES, _LANES)] = rowsum[0]
                    if r + 1 < CH:
                        f_cur, t_cur = f_nxt, t_nxt

                @pl.when(c + NBUF < NCH)
                def _():
                    start_gather(c + NBUF, i)
            return carry

        lax.fori_loop(0, NCH // NBUF, outer, 0)
        pltpu.sync_copy(d2_v, out_hbm.at[pl.ds(pl.multiple_of(base * K, 8), PW)])

    return dist2, CH


_RBL = 2048


def _relayout_body(lo_ref, hi_ref, o_ref):
    o_ref[...] = jnp.concatenate([lo_ref[...].T, hi_ref[...].T], axis=1)


def _relayout(embT, N, D):
    BL = _RBL
    nblk_in = (N + BL - 1) // BL
    nblk = (N + 2 * BL - 1) // (2 * BL)
    return pl.pallas_call(
        _relayout_body,
        grid=(nblk,),
        in_specs=[
            pl.BlockSpec((D, BL),
                         lambda i: (0, jnp.minimum(2 * i, nblk_in - 1))),
            pl.BlockSpec((D, BL),
                         lambda i: (0, jnp.minimum(2 * i + 1, nblk_in - 1))),
        ],
        out_specs=pl.BlockSpec((BL, 2 * D), lambda i: (i, 0)),
        out_shape=jax.ShapeDtypeStruct((nblk * BL, 2 * D), jnp.float32),
    )(embT, embT)


def _row_remap(j):
    blk = j // _RBL
    q = j % _RBL
    return 2 * _RBL * (blk // 2) + 2 * q + (blk % 2)


def _tc_loss_body(d2_ref, v_ref, o_ref):
    x = d2_ref[...]
    v = v_ref[...]
    u = jnp.exp(_B * jnp.log(x))
    m = 1.0 / (1.0 + _A * u)
    att = jnp.log(jnp.clip(m, _EPS, 1.0))
    rep = jnp.log(jnp.clip(1.0 - m, _EPS, 1.0))
    loss = -v * att - _GAMMA * (1.0 - v) * rep
    o_ref[...] = jnp.mean(loss, axis=-1)


def kernel(embedding, V, s, e, J):
    B, K = J.shape
    N, D = embedding.shape
    dist2, CH = _sc_dist2(B, K, D, N)
    start = jnp.minimum(jnp.asarray(s, jnp.int32), jnp.asarray(e, jnp.int32) - B)
    fidx = start + jnp.arange(B, dtype=jnp.int32)
    jr = _row_remap(J)
    fr = _row_remap(fidx)
    j2 = jr.reshape(B * K // CH, CH)
    emb_rm = _relayout(embedding.T, N, D)
    nrows = emb_rm.shape[0] * 2
    emb_lin = emb_rm.reshape(nrows, D)
    d2 = dist2(emb_lin, j2, fr).reshape(B, K)
    return pl.pallas_call(
        _tc_loss_body,
        out_shape=jax.ShapeDtypeStruct((B,), jnp.float32),
    )(d2, V)
